# baseline (speedup 1.0000x reference)
import functools

import jax
import jax.numpy as jnp
from jax import lax
from jax.experimental import pallas as pl
from jax.experimental.pallas import tpu as pltpu
from jax.experimental.pallas import tpu_sc as plsc

N = 10000
D = 128
EMB = 64
E = 320000
NC = 2
NS = 16
NW = NC * NS
L = 16
NPAD = 10112
ROWS_PER_TILE = NPAD // NS
ZCH = 79
EPT = E // NW
CH = 80
NCHUNK = EPT // CH
IDXBLK = 25
NSUPER = NCHUNK // IDXBLK
DEGW = D

_mesh = plsc.VectorSubcoreMesh(core_axis_name="c", subcore_axis_name="s")


@functools.partial(
    pl.kernel,
    mesh=_mesh,
    out_type=jax.ShapeDtypeStruct((NC, NPAD, DEGW), jnp.float32),
    scratch_types=[
        pltpu.VMEM((NSUPER, IDXBLK, CH), jnp.int32),
        pltpu.VMEM((CH, DEGW), jnp.float32),
        pltpu.VMEM_SHARED((NPAD, DEGW), jnp.float32),
    ],
)
def _deg_sc(dst_hbm, out_hbm, dst_v, ones_v, acc):
    c = lax.axis_index("c")
    s = lax.axis_index("s")
    w = c * NS + s

    @pl.loop(0, CH)
    def _zf(i):
        @pl.loop(0, DEGW // L)
        def _zk(k):
            ones_v[i, pl.ds(k * L, L)] = jnp.zeros((L,), jnp.float32)

    @pl.loop(0, ROWS_PER_TILE // ZCH)
    def _zero(t):
        pltpu.sync_copy(ones_v.at[pl.ds(0, ZCH)],
                        acc.at[pl.ds(s * ROWS_PER_TILE + t * ZCH, ZCH)])

    @pl.loop(0, CH)
    def _of(i):
        @pl.loop(0, DEGW // L)
        def _ok(k):
            ones_v[i, pl.ds(k * L, L)] = jnp.ones((L,), jnp.float32)

    pltpu.sync_copy(dst_hbm.at[w], dst_v)
    plsc.subcore_barrier()

    @pl.loop(0, NSUPER)
    def _sb(sb):
        @pl.loop(0, IDXBLK)
        def _scat(j):
            pltpu.sync_copy(ones_v, acc.at[dst_v.at[sb, j]], add=True)

    plsc.subcore_barrier()
    pltpu.sync_copy(
        acc.at[pl.ds(s * ROWS_PER_TILE, ROWS_PER_TILE)],
        out_hbm.at[c, pl.ds(s * ROWS_PER_TILE, ROWS_PER_TILE)],
    )


NBUF = 4


@functools.partial(
    pl.kernel,
    mesh=_mesh,
    out_type=jax.ShapeDtypeStruct((NC, NPAD, D), jnp.float32),
    scratch_types=[
        pltpu.VMEM((IDXBLK, CH), jnp.int32),
        pltpu.VMEM((IDXBLK, CH), jnp.int32),
        pltpu.VMEM((CH, D), jnp.float32),
        pltpu.VMEM((CH, D), jnp.float32),
        pltpu.VMEM((CH, D), jnp.float32),
        pltpu.VMEM((CH, D), jnp.float32),
        pltpu.SemaphoreType.DMA,
        pltpu.SemaphoreType.DMA,
        pltpu.SemaphoreType.DMA,
        pltpu.SemaphoreType.DMA,
        pltpu.SemaphoreType.DMA,
        pltpu.SemaphoreType.DMA,
        pltpu.SemaphoreType.DMA,
        pltpu.SemaphoreType.DMA,
        pltpu.VMEM_SHARED((NPAD, D), jnp.float32),
    ],
)
def _agg_sc(tbl_hbm, src_hbm, dst_hbm, out_hbm, src_v, dst_v, rows0, rows1,
            rows2, rows3, gsem0, gsem1, gsem2, gsem3, ssem0, ssem1, ssem2,
            ssem3, acc):
    c = lax.axis_index("c")
    s = lax.axis_index("s")
    w = c * NS + s
    rows = [rows0, rows1, rows2, rows3]
    gsem = [gsem0, gsem1, gsem2, gsem3]
    ssem = [ssem0, ssem1, ssem2, ssem3]

    @pl.loop(0, CH)
    def _zr(i):
        @pl.loop(0, D // L)
        def _zk(k):
            rows0[i, pl.ds(k * L, L)] = jnp.zeros((L,), jnp.float32)

    @pl.loop(0, ROWS_PER_TILE // ZCH)
    def _zero(t):
        pltpu.sync_copy(rows0.at[pl.ds(0, ZCH)],
                        acc.at[pl.ds(s * ROWS_PER_TILE + t * ZCH, ZCH)])

    plsc.subcore_barrier()

    @pl.loop(0, NSUPER)
    def _super(sb):
        pltpu.sync_copy(src_hbm.at[w, sb], src_v)
        pltpu.sync_copy(dst_hbm.at[w, sb], dst_v)

        gd = [None] * IDXBLK
        sd = [None] * IDXBLK
        for j in range(NBUF):
            gd[j] = pltpu.async_copy(tbl_hbm.at[src_v.at[j]], rows[j], gsem[j])
        for j in range(IDXBLK):
            b = j % NBUF
            gd[j].wait()
            sd[j] = pltpu.async_copy(rows[b], acc.at[dst_v.at[j]], ssem[b],
                                     add=True)
            jn = j + NBUF
            if jn < IDXBLK:
                sd[j].wait()
                gd[jn] = pltpu.async_copy(tbl_hbm.at[src_v.at[jn]], rows[b],
                                          gsem[b])
        for j in range(IDXBLK - NBUF, IDXBLK):
            sd[j].wait()

    plsc.subcore_barrier()
    pltpu.sync_copy(
        acc.at[pl.ds(s * ROWS_PER_TILE, ROWS_PER_TILE)],
        out_hbm.at[c, pl.ds(s * ROWS_PER_TILE, ROWS_PER_TILE)],
    )


BLK = 1000
GRID = N // BLK


def _dinv_from(degp):
    deg = degp[0, :, 0] + degp[1, :, 0] + 1.0
    return lax.rsqrt(deg)[:, None]


def _mm1s_body(x_ref, w_ref, degp_ref, o_ref):
    dinv = _dinv_from(degp_ref[...])
    h = jnp.dot(x_ref[...], w_ref[...], preferred_element_type=jnp.float32,
                precision=lax.Precision.HIGHEST)
    o_ref[...] = h * dinv


def _combine1_body(h1s_ref, p_ref, degp_ref, b1_ref, o_ref):
    dinv = _dinv_from(degp_ref[...])
    h = (p_ref[0] + p_ref[1] + h1s_ref[...]) * dinv + b1_ref[...]
    o_ref[...] = jnp.maximum(h, 0.0) * dinv


def _final_body(hs_ref, q_ref, degp_ref, wcat_ref, bcat_ref, mu_ref, lv_ref):
    dinv = _dinv_from(degp_ref[...])
    agg2 = (q_ref[0] + q_ref[1] + hs_ref[...]) * dinv
    z = jnp.dot(agg2, wcat_ref[...], preferred_element_type=jnp.float32,
                precision=lax.Precision.HIGHEST) + bcat_ref[...]
    mu_ref[...] = z[:, :EMB]
    lv_ref[...] = z[:, EMB:]


def _mm1s(x, W1, degp):
    return pl.pallas_call(
        _mm1s_body,
        grid=(GRID,),
        in_specs=[
            pl.BlockSpec((BLK, D), lambda i: (i, 0)),
            pl.BlockSpec((D, D), lambda i: (0, 0)),
            pl.BlockSpec((NC, BLK, DEGW), lambda i: (0, i, 0)),
        ],
        out_specs=pl.BlockSpec((BLK, D), lambda i: (i, 0)),
        out_shape=jax.ShapeDtypeStruct((N, D), jnp.float32),
    )(x, W1, degp)


def _combine1(h1s, p, degp, b1):
    return pl.pallas_call(
        _combine1_body,
        grid=(GRID,),
        in_specs=[
            pl.BlockSpec((BLK, D), lambda i: (i, 0)),
            pl.BlockSpec((NC, BLK, D), lambda i: (0, i, 0)),
            pl.BlockSpec((NC, BLK, DEGW), lambda i: (0, i, 0)),
            pl.BlockSpec((1, D), lambda i: (0, 0)),
        ],
        out_specs=pl.BlockSpec((BLK, D), lambda i: (i, 0)),
        out_shape=jax.ShapeDtypeStruct((N, D), jnp.float32),
    )(h1s, p, degp, b1)


def _final(hs, q, degp, wcat, bcat):
    return pl.pallas_call(
        _final_body,
        grid=(GRID,),
        in_specs=[
            pl.BlockSpec((BLK, D), lambda i: (i, 0)),
            pl.BlockSpec((NC, BLK, D), lambda i: (0, i, 0)),
            pl.BlockSpec((NC, BLK, DEGW), lambda i: (0, i, 0)),
            pl.BlockSpec((D, 2 * EMB), lambda i: (0, 0)),
            pl.BlockSpec((1, 2 * EMB), lambda i: (0, 0)),
        ],
        out_specs=[
            pl.BlockSpec((BLK, EMB), lambda i: (i, 0)),
            pl.BlockSpec((BLK, EMB), lambda i: (i, 0)),
        ],
        out_shape=[
            jax.ShapeDtypeStruct((N, EMB), jnp.float32),
            jax.ShapeDtypeStruct((N, EMB), jnp.float32),
        ],
    )(hs, q, degp, wcat, bcat)


@jax.jit
def _pipeline(x, edges, W1, b1, Wmu, bmu, Wlv, blv):
    src = edges[0].reshape(NW, NSUPER, IDXBLK, CH)
    dst = edges[1].reshape(NW, NSUPER, IDXBLK, CH)

    degp = _deg_sc(dst)[:, :N, :]
    h1s = _mm1s(x, W1, degp)
    p = _agg_sc(h1s, src, dst)[:, :N, :]
    hs = _combine1(h1s, p, degp, b1.reshape(1, D))
    q = _agg_sc(hs, src, dst)[:, :N, :]
    wcat = jnp.concatenate([Wmu, Wlv], axis=1)
    bcat = jnp.concatenate([bmu, blv]).reshape(1, 2 * EMB)
    return _final(hs, q, degp, wcat, bcat)


def kernel(x, edges, W1, b1, Wmu, bmu, Wlv, blv):
    mu, lv = _pipeline(x, edges.astype(jnp.int32), W1, b1, Wmu, bmu, Wlv, blv)
    return (mu, lv)


# speedup vs baseline: 1.0229x; 1.0229x over previous
import functools

import jax
import jax.numpy as jnp
from jax import lax
from jax.experimental import pallas as pl
from jax.experimental.pallas import tpu as pltpu
from jax.experimental.pallas import tpu_sc as plsc

N = 10000
D = 128
EMB = 64
E = 320000
NC = 2
NS = 16
NW = NC * NS
L = 16
NPAD = 10112
ROWS_PER_TILE = NPAD // NS
ZCH = 79
EPT = E // NW
CH = 80
NCHUNK = EPT // CH
IDXBLK = 25
NSUPER = NCHUNK // IDXBLK
DEGW = D

_mesh = plsc.VectorSubcoreMesh(core_axis_name="c", subcore_axis_name="s")


@functools.partial(
    pl.kernel,
    mesh=_mesh,
    out_type=jax.ShapeDtypeStruct((NC, NPAD, DEGW), jnp.float32),
    scratch_types=[
        pltpu.VMEM((NSUPER, IDXBLK, CH), jnp.int32),
        pltpu.VMEM((CH, DEGW), jnp.float32),
        pltpu.VMEM_SHARED((NPAD, DEGW), jnp.float32),
    ],
)
def _deg_sc(dst_hbm, out_hbm, dst_v, ones_v, acc):
    c = lax.axis_index("c")
    s = lax.axis_index("s")
    w = c * NS + s

    @pl.loop(0, CH)
    def _zf(i):
        @pl.loop(0, DEGW // L)
        def _zk(k):
            ones_v[i, pl.ds(k * L, L)] = jnp.zeros((L,), jnp.float32)

    @pl.loop(0, ROWS_PER_TILE // ZCH)
    def _zero(t):
        pltpu.sync_copy(ones_v.at[pl.ds(0, ZCH)],
                        acc.at[pl.ds(s * ROWS_PER_TILE + t * ZCH, ZCH)])

    @pl.loop(0, CH)
    def _of(i):
        @pl.loop(0, DEGW // L)
        def _ok(k):
            ones_v[i, pl.ds(k * L, L)] = jnp.ones((L,), jnp.float32)

    pltpu.sync_copy(dst_hbm.at[w], dst_v)
    plsc.subcore_barrier()

    @pl.loop(0, NSUPER)
    def _sb(sb):
        @pl.loop(0, IDXBLK)
        def _scat(j):
            pltpu.sync_copy(ones_v, acc.at[dst_v.at[sb, j]], add=True)

    plsc.subcore_barrier()
    pltpu.sync_copy(
        acc.at[pl.ds(s * ROWS_PER_TILE, ROWS_PER_TILE)],
        out_hbm.at[c, pl.ds(s * ROWS_PER_TILE, ROWS_PER_TILE)],
    )


NBUF = 4


@functools.partial(
    pl.kernel,
    mesh=_mesh,
    out_type=jax.ShapeDtypeStruct((NC, NPAD, D), jnp.float32),
    scratch_types=[
        pltpu.VMEM((IDXBLK, CH), jnp.int32),
        pltpu.VMEM((IDXBLK, CH), jnp.int32),
        pltpu.VMEM((CH, D), jnp.float32),
        pltpu.VMEM((CH, D), jnp.float32),
        pltpu.VMEM((CH, D), jnp.float32),
        pltpu.VMEM((CH, D), jnp.float32),
        pltpu.SemaphoreType.DMA,
        pltpu.SemaphoreType.DMA,
        pltpu.SemaphoreType.DMA,
        pltpu.SemaphoreType.DMA,
        pltpu.SemaphoreType.DMA,
        pltpu.SemaphoreType.DMA,
        pltpu.SemaphoreType.DMA,
        pltpu.SemaphoreType.DMA,
        pltpu.VMEM_SHARED((NPAD, D), jnp.float32),
    ],
)
def _agg_sc(tbl_hbm, src_hbm, dst_hbm, out_hbm, src_v, dst_v, rows0, rows1,
            rows2, rows3, gsem0, gsem1, gsem2, gsem3, ssem0, ssem1, ssem2,
            ssem3, acc):
    c = lax.axis_index("c")
    s = lax.axis_index("s")
    w = c * NS + s
    rows = [rows0, rows1, rows2, rows3]
    gsem = [gsem0, gsem1, gsem2, gsem3]
    ssem = [ssem0, ssem1, ssem2, ssem3]

    @pl.loop(0, CH)
    def _zr(i):
        @pl.loop(0, D // L)
        def _zk(k):
            rows0[i, pl.ds(k * L, L)] = jnp.zeros((L,), jnp.float32)

    @pl.loop(0, ROWS_PER_TILE // ZCH)
    def _zero(t):
        pltpu.sync_copy(rows0.at[pl.ds(0, ZCH)],
                        acc.at[pl.ds(s * ROWS_PER_TILE + t * ZCH, ZCH)])

    plsc.subcore_barrier()

    @pl.loop(0, NSUPER)
    def _super(sb):
        pltpu.sync_copy(src_hbm.at[w, sb], src_v)
        pltpu.sync_copy(dst_hbm.at[w, sb], dst_v)

        gd = [None] * IDXBLK
        sd = [None] * IDXBLK
        for j in range(NBUF):
            gd[j] = pltpu.async_copy(tbl_hbm.at[src_v.at[j]], rows[j], gsem[j])
        for j in range(IDXBLK):
            b = j % NBUF
            gd[j].wait()
            sd[j] = pltpu.async_copy(rows[b], acc.at[dst_v.at[j]], ssem[b],
                                     add=True)
            jn = j + NBUF
            if jn < IDXBLK:
                sd[j].wait()
                gd[jn] = pltpu.async_copy(tbl_hbm.at[src_v.at[jn]], rows[b],
                                          gsem[b])
        for j in range(IDXBLK - NBUF, IDXBLK):
            sd[j].wait()

    plsc.subcore_barrier()
    pltpu.sync_copy(
        acc.at[pl.ds(s * ROWS_PER_TILE, ROWS_PER_TILE)],
        out_hbm.at[c, pl.ds(s * ROWS_PER_TILE, ROWS_PER_TILE)],
    )


BLK = 2000
GRID = N // BLK


def _dinv_from(degp):
    deg = degp[0, :, 0] + degp[1, :, 0] + 1.0
    return lax.rsqrt(deg)[:, None]


def _mm1s_body(x_ref, w_ref, degp_ref, o_ref):
    dinv = _dinv_from(degp_ref[...])
    h = jnp.dot(x_ref[...], w_ref[...], preferred_element_type=jnp.float32,
                precision=lax.Precision.HIGHEST)
    o_ref[...] = h * dinv


def _combine1_body(h1s_ref, p_ref, degp_ref, b1_ref, o_ref):
    dinv = _dinv_from(degp_ref[...])
    h = (p_ref[0] + p_ref[1] + h1s_ref[...]) * dinv + b1_ref[...]
    o_ref[...] = jnp.maximum(h, 0.0) * dinv


def _final_body(hs_ref, q_ref, degp_ref, wcat_ref, bcat_ref, mu_ref, lv_ref):
    dinv = _dinv_from(degp_ref[...])
    agg2 = (q_ref[0] + q_ref[1] + hs_ref[...]) * dinv
    z = jnp.dot(agg2, wcat_ref[...], preferred_element_type=jnp.float32,
                precision=lax.Precision.HIGHEST) + bcat_ref[...]
    mu_ref[...] = z[:, :EMB]
    lv_ref[...] = z[:, EMB:]


def _mm1s(x, W1, degp):
    return pl.pallas_call(
        _mm1s_body,
        grid=(GRID,),
        in_specs=[
            pl.BlockSpec((BLK, D), lambda i: (i, 0)),
            pl.BlockSpec((D, D), lambda i: (0, 0)),
            pl.BlockSpec((NC, BLK, DEGW), lambda i: (0, i, 0)),
        ],
        out_specs=pl.BlockSpec((BLK, D), lambda i: (i, 0)),
        out_shape=jax.ShapeDtypeStruct((N, D), jnp.float32),
    )(x, W1, degp)


def _combine1(h1s, p, degp, b1):
    return pl.pallas_call(
        _combine1_body,
        grid=(GRID,),
        in_specs=[
            pl.BlockSpec((BLK, D), lambda i: (i, 0)),
            pl.BlockSpec((NC, BLK, D), lambda i: (0, i, 0)),
            pl.BlockSpec((NC, BLK, DEGW), lambda i: (0, i, 0)),
            pl.BlockSpec((1, D), lambda i: (0, 0)),
        ],
        out_specs=pl.BlockSpec((BLK, D), lambda i: (i, 0)),
        out_shape=jax.ShapeDtypeStruct((N, D), jnp.float32),
    )(h1s, p, degp, b1)


def _final(hs, q, degp, wcat, bcat):
    return pl.pallas_call(
        _final_body,
        grid=(GRID,),
        in_specs=[
            pl.BlockSpec((BLK, D), lambda i: (i, 0)),
            pl.BlockSpec((NC, BLK, D), lambda i: (0, i, 0)),
            pl.BlockSpec((NC, BLK, DEGW), lambda i: (0, i, 0)),
            pl.BlockSpec((D, 2 * EMB), lambda i: (0, 0)),
            pl.BlockSpec((1, 2 * EMB), lambda i: (0, 0)),
        ],
        out_specs=[
            pl.BlockSpec((BLK, EMB), lambda i: (i, 0)),
            pl.BlockSpec((BLK, EMB), lambda i: (i, 0)),
        ],
        out_shape=[
            jax.ShapeDtypeStruct((N, EMB), jnp.float32),
            jax.ShapeDtypeStruct((N, EMB), jnp.float32),
        ],
    )(hs, q, degp, wcat, bcat)


@jax.jit
def _pipeline(x, edges, W1, b1, Wmu, bmu, Wlv, blv):
    src = edges[0].reshape(NW, NSUPER, IDXBLK, CH)
    dst = edges[1].reshape(NW, NSUPER, IDXBLK, CH)

    degp = _deg_sc(dst)[:, :N, :]
    h1s = _mm1s(x, W1, degp)
    p = _agg_sc(h1s, src, dst)[:, :N, :]
    hs = _combine1(h1s, p, degp, b1.reshape(1, D))
    q = _agg_sc(hs, src, dst)[:, :N, :]
    wcat = jnp.concatenate([Wmu, Wlv], axis=1)
    bcat = jnp.concatenate([bmu, blv]).reshape(1, 2 * EMB)
    return _final(hs, q, degp, wcat, bcat)


def kernel(x, edges, W1, b1, Wmu, bmu, Wlv, blv):
    mu, lv = _pipeline(x, edges.astype(jnp.int32), W1, b1, Wmu, bmu, Wlv, blv)
    return (mu, lv)
